# baseline (speedup 1.0000x reference)
import functools

import jax
import jax.numpy as jnp
from jax import lax
from jax.experimental import pallas as pl
from jax.experimental.pallas import tpu as pltpu
from jax.experimental.pallas import tpu_sc as plsc

N = 10000
E = 320000
D = 128
NUM_REL = 10
NUM_LAYERS = 3
NUM_HEADS = 4
HEAD_DIM = D // NUM_HEADS

NC = 2
NS = 16
NW = NC * NS
CHUNK = 64
EPW = 10240
EPAD = NW * EPW
NPAD = 10112
BM = 1000


def _ln(x, g, b):
    m = jnp.mean(x, axis=-1, keepdims=True)
    v = jnp.mean((x - m) ** 2, axis=-1, keepdims=True)
    return (x - m) * lax.rsqrt(v + 1e-5) * g + b


def _transform(x, rW_ref, rb_ref, sW_ref, sb_ref, h_ref, self_ref):
    for r in range(NUM_REL):
        h_ref[r] = (
            jnp.dot(x, rW_ref[r], preferred_element_type=jnp.float32)
            + rb_ref[r][None, :]
        )
    self_ref[...] = (
        jnp.dot(x, sW_ref[...], preferred_element_type=jnp.float32)
        + sb_ref[0][None, :]
    )


def _t0_body(types_ref, nemb_ref, temb_ref, rW_ref, rb_ref, sW_ref, sb_ref,
             h_ref, self_ref):
    t = types_ref[0]
    x = nemb_ref[...]
    for tt in range(10):
        row = temb_ref[tt][None, :]
        x = x + jnp.where(t == tt, 1.0, 0.0) * row
    _transform(x, rW_ref, rb_ref, sW_ref, sb_ref, h_ref, self_ref)


def _t0(node_types, node_emb, type_emb, rW, rb, sW, sb):
    types3 = node_types.astype(jnp.int32).reshape(N // BM, BM, 1)
    h, self_out = pl.pallas_call(
        _t0_body,
        grid=(N // BM,),
        in_specs=[
            pl.BlockSpec((1, BM, 1), lambda i: (i, 0, 0)),
            pl.BlockSpec((BM, D), lambda i: (i, 0)),
            pl.BlockSpec((10, D), lambda i: (0, 0)),
            pl.BlockSpec((NUM_REL, D, D), lambda i: (0, 0, 0)),
            pl.BlockSpec((NUM_REL, D), lambda i: (0, 0)),
            pl.BlockSpec((D, D), lambda i: (0, 0)),
            pl.BlockSpec((1, D), lambda i: (0, 0)),
        ],
        out_specs=[
            pl.BlockSpec((NUM_REL, BM, D), lambda i: (0, i, 0)),
            pl.BlockSpec((BM, D), lambda i: (i, 0)),
        ],
        out_shape=[
            jax.ShapeDtypeStruct((NUM_REL, N, D), jnp.float32),
            jax.ShapeDtypeStruct((N, D), jnp.float32),
        ],
    )(types3, node_emb, type_emb, rW, rb, sW.reshape(D, D), sb.reshape(1, D))
    return h.reshape(NUM_REL * N, D), self_out


def _cidx_body(rel_ref, src_ref, out_ref):
    out_ref[...] = rel_ref[...] * N + src_ref[...]


def _make_cidx(rel_p, src_p):
    rel2 = rel_p.reshape(EPAD // CHUNK, CHUNK)
    src2 = src_p.reshape(EPAD // CHUNK, CHUNK)
    return pl.pallas_call(
        _cidx_body,
        in_specs=[pl.BlockSpec(rel2.shape, lambda: (0, 0)),
                  pl.BlockSpec(src2.shape, lambda: (0, 0))],
        out_specs=pl.BlockSpec(rel2.shape, lambda: (0, 0)),
        out_shape=jax.ShapeDtypeStruct(rel2.shape, jnp.int32),
    )(rel2, src2)


def _post_val(s_ref, p_ref, xp, g1_ref, b1_ref, g2_ref, b2_ref):
    out = s_ref[...] + p_ref[0] + p_ref[1]
    h = jnp.maximum(_ln(out, g1_ref[...], b1_ref[...]), 0.0)
    if xp is not None:
        h = h + xp
    return _ln(h, g2_ref[...], b2_ref[...])


def _fmid_body_resid(s_ref, p_ref, xp_ref, g1_ref, b1_ref, g2_ref, b2_ref,
                     rW_ref, rb_ref, sW_ref, sb_ref, x_ref, h_ref, self_ref):
    x = _post_val(s_ref, p_ref, xp_ref[...], g1_ref, b1_ref, g2_ref, b2_ref)
    x_ref[...] = x
    _transform(x, rW_ref, rb_ref, sW_ref, sb_ref, h_ref, self_ref)


def _fmid_body_noresid(s_ref, p_ref, g1_ref, b1_ref, g2_ref, b2_ref,
                       rW_ref, rb_ref, sW_ref, sb_ref, x_ref, h_ref,
                       self_ref):
    x = _post_val(s_ref, p_ref, None, g1_ref, b1_ref, g2_ref, b2_ref)
    x_ref[...] = x
    _transform(x, rW_ref, rb_ref, sW_ref, sb_ref, h_ref, self_ref)


def _fmid(self_out, parts, x_prev, g1, b1, g2, b2, rW, rb, sW, sb,
          with_resid):
    row = pl.BlockSpec((BM, D), lambda i: (i, 0))
    pspec = pl.BlockSpec((NC, BM, D), lambda i: (0, i, 0))
    vec = pl.BlockSpec((1, D), lambda i: (0, 0))
    if with_resid:
        body = _fmid_body_resid
        ops = [self_out, parts, x_prev]
        specs = [row, pspec, row]
    else:
        body = _fmid_body_noresid
        ops = [self_out, parts]
        specs = [row, pspec]
    wspecs = [
        pl.BlockSpec((NUM_REL, D, D), lambda i: (0, 0, 0)),
        pl.BlockSpec((NUM_REL, D), lambda i: (0, 0)),
        pl.BlockSpec((D, D), lambda i: (0, 0)),
        pl.BlockSpec((1, D), lambda i: (0, 0)),
    ]
    x, h, self_o = pl.pallas_call(
        body,
        grid=(N // BM,),
        in_specs=specs + [vec] * 4 + wspecs,
        out_specs=[
            row,
            pl.BlockSpec((NUM_REL, BM, D), lambda i: (0, i, 0)),
            row,
        ],
        out_shape=[
            jax.ShapeDtypeStruct((N, D), jnp.float32),
            jax.ShapeDtypeStruct((NUM_REL, N, D), jnp.float32),
            jax.ShapeDtypeStruct((N, D), jnp.float32),
        ],
    )(*ops, g1.reshape(1, D), b1.reshape(1, D), g2.reshape(1, D),
      b2.reshape(1, D), rW, rb, sW.reshape(D, D), sb.reshape(1, D))
    return x, h.reshape(NUM_REL * N, D), self_o


def _proj_body(x_ref, qW_ref, qb_ref, kW_ref, kb_ref, vW_ref, vb_ref,
               q_ref, k_ref, v_ref):
    x = x_ref[...]
    q_ref[...] = jnp.dot(x, qW_ref[...],
                         preferred_element_type=jnp.float32) + qb_ref[...]
    k_ref[...] = jnp.dot(x, kW_ref[...],
                         preferred_element_type=jnp.float32) + kb_ref[...]
    v_ref[...] = jnp.dot(x, vW_ref[...],
                         preferred_element_type=jnp.float32) + vb_ref[...]


def _proj(x, qW, qb, kW, kb, vW, vb):
    row = pl.BlockSpec((BM, D), lambda i: (i, 0))
    mat = pl.BlockSpec((D, D), lambda i: (0, 0))
    vec = pl.BlockSpec((1, D), lambda i: (0, 0))
    return pl.pallas_call(
        _proj_body,
        grid=(N // BM,),
        in_specs=[row, mat, vec, mat, vec, mat, vec],
        out_specs=[row, row, row],
        out_shape=[jax.ShapeDtypeStruct((N, D), jnp.float32)] * 3,
    )(x, qW, qb.reshape(1, D), kW, kb.reshape(1, D), vW, vb.reshape(1, D))


def _readout_body(s_ref, p_ref, x2_ref, q1_ref, k1_ref, v1_ref, q2_ref,
                  k2_ref, v2_ref, g1_ref, b1_ref, g2_ref,
                  b2_ref, qW_ref, qb_ref, kW_ref, kb_ref,
                  vW_ref, vb_ref, oW_ref, ob_ref, p1W_ref, p1b_ref, lg_ref,
                  lb_ref, p2W_ref, p2b_ref, out_ref):
    x3 = _post_val(s_ref, p_ref, x2_ref[...], g1_ref, b1_ref, g2_ref, b2_ref)
    q = [q1_ref[...], q2_ref[...],
         jnp.dot(x3, qW_ref[...], preferred_element_type=jnp.float32)
         + qb_ref[...]]
    k = [k1_ref[...], k2_ref[...],
         jnp.dot(x3, kW_ref[...], preferred_element_type=jnp.float32)
         + kb_ref[...]]
    v = [v1_ref[...], v2_ref[...],
         jnp.dot(x3, vW_ref[...], preferred_element_type=jnp.float32)
         + vb_ref[...]]
    di = lax.broadcasted_iota(jnp.int32, (D, D), 0) // HEAD_DIM
    ei = lax.broadcasted_iota(jnp.int32, (D, D), 1) // HEAD_DIM
    msum = jnp.where(di == ei, 1.0 / (HEAD_DIM ** 0.5), 0.0)
    logit = [[jnp.dot(q[i] * k[j], msum, preferred_element_type=jnp.float32)
              for j in range(NUM_LAYERS)] for i in range(NUM_LAYERS)]
    om = jnp.zeros_like(q[0])
    for i in range(NUM_LAYERS):
        li = logit[i]
        m = jnp.maximum(jnp.maximum(li[0], li[1]), li[2])
        e = [jnp.exp(l - m) for l in li]
        inv = 1.0 / (e[0] + e[1] + e[2])
        om = om + (e[0] * v[0] + e[1] * v[1] + e[2] * v[2]) * inv
    om = om * (1.0 / NUM_LAYERS)
    xm = jnp.dot(om, oW_ref[...], preferred_element_type=jnp.float32) + ob_ref[...]
    h1 = jnp.dot(xm, p1W_ref[...], preferred_element_type=jnp.float32) + p1b_ref[...]
    h1 = jnp.maximum(_ln(h1, lg_ref[...], lb_ref[...]), 0.0)
    out_ref[...] = jnp.dot(h1, p2W_ref[...],
                           preferred_element_type=jnp.float32) + p2b_ref[...]


def _readout(self_out, parts, x2, qkv1, qkv2, g1, b1, g2, b2, qW, qb, kW, kb,
             vW, vb, oW, ob, p1W, p1b, lg, lb, p2W, p2b):
    row = pl.BlockSpec((BM, D), lambda i: (i, 0))
    pspec = pl.BlockSpec((NC, BM, D), lambda i: (0, i, 0))
    mat = pl.BlockSpec((D, D), lambda i: (0, 0))
    vec = pl.BlockSpec((1, D), lambda i: (0, 0))
    return pl.pallas_call(
        _readout_body,
        grid=(N // BM,),
        in_specs=[row, pspec, row] + [row] * 6 + [vec, vec, vec, vec,
                  mat, vec, mat, vec, mat, vec, mat, vec, mat,
                  vec, vec, vec, mat, vec],
        out_specs=row,
        out_shape=jax.ShapeDtypeStruct((N, D), jnp.float32),
    )(self_out, parts, x2, *qkv1, *qkv2, g1.reshape(1, D), b1.reshape(1, D),
      g2.reshape(1, D), b2.reshape(1, D),
      qW, qb.reshape(1, D), kW, kb.reshape(1, D), vW,
      vb.reshape(1, D), oW, ob.reshape(1, D), p1W, p1b.reshape(1, D),
      lg.reshape(1, D), lb.reshape(1, D), p2W, p2b.reshape(1, D))


NCH = EPW // CHUNK
NH = NCH // 2
NBUF = 3


def _sc_scatter(h_flat, cidx2, dst2, zeros):
    mesh = plsc.VectorSubcoreMesh(core_axis_name="c", subcore_axis_name="s")

    @functools.partial(
        pl.kernel,
        out_type=jax.ShapeDtypeStruct((NC, NPAD, D), jnp.float32),
        mesh=mesh,
        scratch_types=[
            pltpu.VMEM_SHARED((NPAD, D), jnp.float32),
            pltpu.VMEM((NH, CHUNK), jnp.int32),
            pltpu.VMEM((NH, CHUNK), jnp.int32),
        ] + [pltpu.VMEM((CHUNK, D), jnp.float32)] * NBUF
          + [pltpu.SemaphoreType.DMA] * (2 * NBUF),
    )
    def k(h_hbm, ci_hbm, di_hbm, z_hbm, out_hbm, acc, gi_all, di_all,
          r0, r1, r2, sg0, sg1, sg2, ss0, ss1, ss2):
        rows = [r0, r1, r2]
        sg = [sg0, sg1, sg2]
        ss = [ss0, ss1, ss2]
        c = lax.axis_index("c")
        s = lax.axis_index("s")
        zr = NPAD // NS
        pltpu.sync_copy(z_hbm.at[pl.ds(s * zr, zr)], acc.at[pl.ds(s * zr, zr)])
        wrow = (c * NS + s) * NCH
        plsc.subcore_barrier()

        def gath(j, b):
            pltpu.async_copy(h_hbm.at[gi_all.at[j]], rows[b], sg[b])

        def wait_g(b):
            pltpu.make_async_copy(h_hbm.at[gi_all.at[0]], rows[b],
                                  sg[b]).wait()

        def scat(j, b):
            pltpu.async_copy(rows[b], acc.at[di_all.at[j]], ss[b], add=True)

        def wait_s(b):
            pltpu.make_async_copy(rows[b], acc.at[di_all.at[0]],
                                  ss[b]).wait()

        for half in range(2):
            pltpu.sync_copy(ci_hbm.at[pl.ds(wrow + half * NH, NH)], gi_all)
            pltpu.sync_copy(di_hbm.at[pl.ds(wrow + half * NH, NH)], di_all)
            gath(0, 0)
            for j in range(2):
                wait_g(j % NBUF)
                scat(j, j % NBUF)
                gath(j + 1, (j + 1) % NBUF)

            @pl.loop(0, (NH - 5) // NBUF)
            def _(t):
                j0 = 2 + t * NBUF
                for u in range(NBUF):
                    j = j0 + u
                    b = (2 + u) % NBUF
                    wait_g(b)
                    scat(j, b)
                    wait_s((b + 1) % NBUF)
                    gath(j + 1, (b + 1) % NBUF)

            for j in range(NH - 3, NH):
                b = j % NBUF
                wait_g(b)
                scat(j, b)
                if j < NH - 1:
                    wait_s((b + 1) % NBUF)
                    gath(j + 1, (b + 1) % NBUF)
            for b in range(NBUF):
                wait_s(b)

        plsc.subcore_barrier()
        pltpu.sync_copy(acc.at[pl.ds(s * zr, zr)],
                        out_hbm.at[c].at[pl.ds(s * zr, zr)])

    return k(h_flat, cidx2, dst2, zeros)


def kernel(node_ids, node_types, edge_index, edge_type, node_emb, type_emb,
           rel_W, rel_b, self_W, self_b, conv_ln_g, conv_ln_b, norm_g, norm_b,
           qW, qb, kW, kb, vW, vb, oW, ob, op1_W, op1_b, op_ln_g, op_ln_b,
           op2_W, op2_b):
    src = edge_index[0].astype(jnp.int32)
    dst = edge_index[1].astype(jnp.int32)
    rel = edge_type.astype(jnp.int32)

    pad = EPAD - E
    pad_ar = jnp.arange(pad, dtype=jnp.int32)
    src_p = jnp.concatenate([src, pad_ar % 128])
    rel_p = jnp.concatenate([rel, jnp.zeros((pad,), jnp.int32)])
    dst_p = jnp.concatenate([dst, N + pad_ar % (NPAD - N)]).reshape(
        EPAD // CHUNK, CHUNK)
    cidx = _make_cidx(rel_p, src_p)
    zeros = jnp.zeros((NPAD, D), jnp.float32)

    h0, self0 = _t0(node_types, node_emb, type_emb, rel_W[0], rel_b[0],
                    self_W[0], self_b[0])
    parts0 = _sc_scatter(h0, cidx, dst_p, zeros)
    x1, h1, self1 = _fmid(self0, parts0, None, conv_ln_g[0], conv_ln_b[0],
                          norm_g[0], norm_b[0], rel_W[1], rel_b[1],
                          self_W[1], self_b[1], with_resid=False)
    parts1 = _sc_scatter(h1, cidx, dst_p, zeros)
    qkv1 = _proj(x1, qW, qb, kW, kb, vW, vb)
    x2, h2, self2 = _fmid(self1, parts1, x1, conv_ln_g[1], conv_ln_b[1],
                          norm_g[1], norm_b[1], rel_W[2], rel_b[2],
                          self_W[2], self_b[2], with_resid=True)
    parts2 = _sc_scatter(h2, cidx, dst_p, zeros)
    qkv2 = _proj(x2, qW, qb, kW, kb, vW, vb)
    return _readout(self2, parts2, x2, qkv1, qkv2, conv_ln_g[2],
                    conv_ln_b[2], norm_g[2], norm_b[2], qW, qb, kW, kb, vW,
                    vb, oW, ob, op1_W, op1_b, op_ln_g, op_ln_b, op2_W, op2_b)


# speedup vs baseline: 1.4662x; 1.4662x over previous
import functools

import jax
import jax.numpy as jnp
from jax import lax
from jax.experimental import pallas as pl
from jax.experimental.pallas import tpu as pltpu
from jax.experimental.pallas import tpu_sc as plsc

N = 10000
E = 320000
D = 128
NUM_REL = 10
NUM_LAYERS = 3
NUM_HEADS = 4
HEAD_DIM = D // NUM_HEADS

NC = 2
NS = 16
NW = NC * NS
CHUNK = 128
EPW = 10240
EPAD = NW * EPW
NPAD = 10112
BM = 2000


def _ln(x, g, b):
    m = jnp.mean(x, axis=-1, keepdims=True)
    v = jnp.mean((x - m) ** 2, axis=-1, keepdims=True)
    return (x - m) * lax.rsqrt(v + 1e-5) * g + b


def _transform(x, rW_ref, rb_ref, sW_ref, sb_ref, h_ref, self_ref):
    for r in range(NUM_REL):
        h_ref[r] = (
            jnp.dot(x, rW_ref[r], preferred_element_type=jnp.float32)
            + rb_ref[r][None, :]
        )
    self_ref[...] = (
        jnp.dot(x, sW_ref[...], preferred_element_type=jnp.float32)
        + sb_ref[0][None, :]
    )


def _t0_body(types_ref, nemb_ref, temb_ref, rW_ref, rb_ref, sW_ref, sb_ref,
             h_ref, self_ref):
    t = types_ref[0]
    x = nemb_ref[...]
    for tt in range(10):
        row = temb_ref[tt][None, :]
        x = x + jnp.where(t == tt, 1.0, 0.0) * row
    _transform(x, rW_ref, rb_ref, sW_ref, sb_ref, h_ref, self_ref)


def _t0(node_types, node_emb, type_emb, rW, rb, sW, sb):
    types3 = node_types.astype(jnp.int32).reshape(N // BM, BM, 1)
    h, self_out = pl.pallas_call(
        _t0_body,
        grid=(N // BM,),
        in_specs=[
            pl.BlockSpec((1, BM, 1), lambda i: (i, 0, 0)),
            pl.BlockSpec((BM, D), lambda i: (i, 0)),
            pl.BlockSpec((10, D), lambda i: (0, 0)),
            pl.BlockSpec((NUM_REL, D, D), lambda i: (0, 0, 0)),
            pl.BlockSpec((NUM_REL, D), lambda i: (0, 0)),
            pl.BlockSpec((D, D), lambda i: (0, 0)),
            pl.BlockSpec((1, D), lambda i: (0, 0)),
        ],
        out_specs=[
            pl.BlockSpec((NUM_REL, BM, D), lambda i: (0, i, 0)),
            pl.BlockSpec((BM, D), lambda i: (i, 0)),
        ],
        out_shape=[
            jax.ShapeDtypeStruct((NUM_REL, N, D), jnp.float32),
            jax.ShapeDtypeStruct((N, D), jnp.float32),
        ],
    )(types3, node_emb, type_emb, rW, rb, sW.reshape(D, D), sb.reshape(1, D))
    return h.reshape(NUM_REL * N, D), self_out


def _cidx_body(rel_ref, src_ref, out_ref):
    out_ref[...] = rel_ref[...] * N + src_ref[...]


def _make_cidx(rel_p, src_p):
    rel2 = rel_p.reshape(EPAD // 128, 128)
    src2 = src_p.reshape(EPAD // 128, 128)
    return pl.pallas_call(
        _cidx_body,
        in_specs=[pl.BlockSpec(rel2.shape, lambda: (0, 0)),
                  pl.BlockSpec(src2.shape, lambda: (0, 0))],
        out_specs=pl.BlockSpec(rel2.shape, lambda: (0, 0)),
        out_shape=jax.ShapeDtypeStruct(rel2.shape, jnp.int32),
    )(rel2, src2)


def _post_val(s_ref, p_ref, xp, g1_ref, b1_ref, g2_ref, b2_ref):
    out = s_ref[...] + p_ref[0] + p_ref[1]
    h = jnp.maximum(_ln(out, g1_ref[...], b1_ref[...]), 0.0)
    if xp is not None:
        h = h + xp
    return _ln(h, g2_ref[...], b2_ref[...])


def _fmid_body_resid(s_ref, p_ref, xp_ref, g1_ref, b1_ref, g2_ref, b2_ref,
                     rW_ref, rb_ref, sW_ref, sb_ref, x_ref, h_ref, self_ref):
    x = _post_val(s_ref, p_ref, xp_ref[...], g1_ref, b1_ref, g2_ref, b2_ref)
    x_ref[...] = x
    _transform(x, rW_ref, rb_ref, sW_ref, sb_ref, h_ref, self_ref)


def _fmid_body_noresid(s_ref, p_ref, g1_ref, b1_ref, g2_ref, b2_ref,
                       rW_ref, rb_ref, sW_ref, sb_ref, x_ref, h_ref,
                       self_ref):
    x = _post_val(s_ref, p_ref, None, g1_ref, b1_ref, g2_ref, b2_ref)
    x_ref[...] = x
    _transform(x, rW_ref, rb_ref, sW_ref, sb_ref, h_ref, self_ref)


def _fmid(self_out, parts, x_prev, g1, b1, g2, b2, rW, rb, sW, sb,
          with_resid):
    row = pl.BlockSpec((BM, D), lambda i: (i, 0))
    pspec = pl.BlockSpec((NC, BM, D), lambda i: (0, i, 0))
    vec = pl.BlockSpec((1, D), lambda i: (0, 0))
    if with_resid:
        body = _fmid_body_resid
        ops = [self_out, parts, x_prev]
        specs = [row, pspec, row]
    else:
        body = _fmid_body_noresid
        ops = [self_out, parts]
        specs = [row, pspec]
    wspecs = [
        pl.BlockSpec((NUM_REL, D, D), lambda i: (0, 0, 0)),
        pl.BlockSpec((NUM_REL, D), lambda i: (0, 0)),
        pl.BlockSpec((D, D), lambda i: (0, 0)),
        pl.BlockSpec((1, D), lambda i: (0, 0)),
    ]
    x, h, self_o = pl.pallas_call(
        body,
        grid=(N // BM,),
        in_specs=specs + [vec] * 4 + wspecs,
        out_specs=[
            row,
            pl.BlockSpec((NUM_REL, BM, D), lambda i: (0, i, 0)),
            row,
        ],
        out_shape=[
            jax.ShapeDtypeStruct((N, D), jnp.float32),
            jax.ShapeDtypeStruct((NUM_REL, N, D), jnp.float32),
            jax.ShapeDtypeStruct((N, D), jnp.float32),
        ],
    )(*ops, g1.reshape(1, D), b1.reshape(1, D), g2.reshape(1, D),
      b2.reshape(1, D), rW, rb, sW.reshape(D, D), sb.reshape(1, D))
    return x, h.reshape(NUM_REL * N, D), self_o


def _proj_body(x_ref, qW_ref, qb_ref, kW_ref, kb_ref, vW_ref, vb_ref,
               q_ref, k_ref, v_ref):
    x = x_ref[...]
    q_ref[...] = jnp.dot(x, qW_ref[...],
                         preferred_element_type=jnp.float32) + qb_ref[...]
    k_ref[...] = jnp.dot(x, kW_ref[...],
                         preferred_element_type=jnp.float32) + kb_ref[...]
    v_ref[...] = jnp.dot(x, vW_ref[...],
                         preferred_element_type=jnp.float32) + vb_ref[...]


def _proj(x, qW, qb, kW, kb, vW, vb):
    row = pl.BlockSpec((BM, D), lambda i: (i, 0))
    mat = pl.BlockSpec((D, D), lambda i: (0, 0))
    vec = pl.BlockSpec((1, D), lambda i: (0, 0))
    return pl.pallas_call(
        _proj_body,
        grid=(N // BM,),
        in_specs=[row, mat, vec, mat, vec, mat, vec],
        out_specs=[row, row, row],
        out_shape=[jax.ShapeDtypeStruct((N, D), jnp.float32)] * 3,
    )(x, qW, qb.reshape(1, D), kW, kb.reshape(1, D), vW, vb.reshape(1, D))


def _readout_body(s_ref, p_ref, x2_ref, q1_ref, k1_ref, v1_ref, q2_ref,
                  k2_ref, v2_ref, g1_ref, b1_ref, g2_ref,
                  b2_ref, qW_ref, qb_ref, kW_ref, kb_ref,
                  vW_ref, vb_ref, oW_ref, ob_ref, p1W_ref, p1b_ref, lg_ref,
                  lb_ref, p2W_ref, p2b_ref, out_ref):
    x3 = _post_val(s_ref, p_ref, x2_ref[...], g1_ref, b1_ref, g2_ref, b2_ref)
    q = [q1_ref[...], q2_ref[...],
         jnp.dot(x3, qW_ref[...], preferred_element_type=jnp.float32)
         + qb_ref[...]]
    k = [k1_ref[...], k2_ref[...],
         jnp.dot(x3, kW_ref[...], preferred_element_type=jnp.float32)
         + kb_ref[...]]
    v = [v1_ref[...], v2_ref[...],
         jnp.dot(x3, vW_ref[...], preferred_element_type=jnp.float32)
         + vb_ref[...]]
    di = lax.broadcasted_iota(jnp.int32, (D, D), 0) // HEAD_DIM
    ei = lax.broadcasted_iota(jnp.int32, (D, D), 1) // HEAD_DIM
    msum = jnp.where(di == ei, 1.0 / (HEAD_DIM ** 0.5), 0.0)
    logit = [[jnp.dot(q[i] * k[j], msum, preferred_element_type=jnp.float32)
              for j in range(NUM_LAYERS)] for i in range(NUM_LAYERS)]
    om = jnp.zeros_like(q[0])
    for i in range(NUM_LAYERS):
        li = logit[i]
        m = jnp.maximum(jnp.maximum(li[0], li[1]), li[2])
        e = [jnp.exp(l - m) for l in li]
        inv = 1.0 / (e[0] + e[1] + e[2])
        om = om + (e[0] * v[0] + e[1] * v[1] + e[2] * v[2]) * inv
    om = om * (1.0 / NUM_LAYERS)
    xm = jnp.dot(om, oW_ref[...], preferred_element_type=jnp.float32) + ob_ref[...]
    h1 = jnp.dot(xm, p1W_ref[...], preferred_element_type=jnp.float32) + p1b_ref[...]
    h1 = jnp.maximum(_ln(h1, lg_ref[...], lb_ref[...]), 0.0)
    out_ref[...] = jnp.dot(h1, p2W_ref[...],
                           preferred_element_type=jnp.float32) + p2b_ref[...]


def _readout(self_out, parts, x2, qkv1, qkv2, g1, b1, g2, b2, qW, qb, kW, kb,
             vW, vb, oW, ob, p1W, p1b, lg, lb, p2W, p2b):
    row = pl.BlockSpec((BM, D), lambda i: (i, 0))
    pspec = pl.BlockSpec((NC, BM, D), lambda i: (0, i, 0))
    mat = pl.BlockSpec((D, D), lambda i: (0, 0))
    vec = pl.BlockSpec((1, D), lambda i: (0, 0))
    return pl.pallas_call(
        _readout_body,
        grid=(N // BM,),
        in_specs=[row, pspec, row] + [row] * 6 + [vec, vec, vec, vec,
                  mat, vec, mat, vec, mat, vec, mat, vec, mat,
                  vec, vec, vec, mat, vec],
        out_specs=row,
        out_shape=jax.ShapeDtypeStruct((N, D), jnp.float32),
    )(self_out, parts, x2, *qkv1, *qkv2, g1.reshape(1, D), b1.reshape(1, D),
      g2.reshape(1, D), b2.reshape(1, D),
      qW, qb.reshape(1, D), kW, kb.reshape(1, D), vW,
      vb.reshape(1, D), oW, ob.reshape(1, D), p1W, p1b.reshape(1, D),
      lg.reshape(1, D), lb.reshape(1, D), p2W, p2b.reshape(1, D))


NCH = EPW // CHUNK


def _sc_scatter(h_flat, cidx2, dst2, zeros):
    mesh = plsc.VectorSubcoreMesh(core_axis_name="c", subcore_axis_name="s")

    @functools.partial(
        pl.kernel,
        out_type=jax.ShapeDtypeStruct((NC, NPAD, D), jnp.float32),
        mesh=mesh,
        scratch_types=[
            pltpu.VMEM_SHARED((NPAD, D), jnp.float32),
            pltpu.VMEM((NCH // 2, CHUNK), jnp.int32),
            pltpu.VMEM((NCH // 2, CHUNK), jnp.int32),
            pltpu.VMEM((CHUNK, D), jnp.float32),
            pltpu.VMEM((CHUNK, D), jnp.float32),
            pltpu.SemaphoreType.DMA,
            pltpu.SemaphoreType.DMA,
        ],
    )
    def k(h_hbm, ci_hbm, di_hbm, z_hbm, out_hbm, acc, gi_all, di_all,
          rows0, rows1, sem0, sem1):
        c = lax.axis_index("c")
        s = lax.axis_index("s")
        zr = NPAD // NS
        pltpu.sync_copy(z_hbm.at[pl.ds(s * zr, zr)], acc.at[pl.ds(s * zr, zr)])
        wrow = (c * NS + s) * NCH
        plsc.subcore_barrier()

        nh = NCH // 2
        for half in range(2):
            pltpu.sync_copy(ci_hbm.at[pl.ds(wrow + half * nh, nh)], gi_all)
            pltpu.sync_copy(di_hbm.at[pl.ds(wrow + half * nh, nh)], di_all)
            pltpu.async_copy(h_hbm.at[gi_all.at[0]], rows0, sem0)

            @pl.loop(0, nh // 2)
            def _(jj):
                j = jj * 2
                pltpu.async_copy(h_hbm.at[gi_all.at[j + 1]], rows1, sem1)
                pltpu.make_async_copy(h_hbm.at[gi_all.at[0]], rows0,
                                      sem0).wait()
                pltpu.sync_copy(rows0, acc.at[di_all.at[j]], add=True)

                @pl.when(jj < nh // 2 - 1)
                def _():
                    pltpu.async_copy(h_hbm.at[gi_all.at[j + 2]], rows0, sem0)

                pltpu.make_async_copy(h_hbm.at[gi_all.at[0]], rows1,
                                      sem1).wait()
                pltpu.sync_copy(rows1, acc.at[di_all.at[j + 1]], add=True)

        plsc.subcore_barrier()
        pltpu.sync_copy(acc.at[pl.ds(s * zr, zr)],
                        out_hbm.at[c].at[pl.ds(s * zr, zr)])

    return k(h_flat, cidx2, dst2, zeros)


def kernel(node_ids, node_types, edge_index, edge_type, node_emb, type_emb,
           rel_W, rel_b, self_W, self_b, conv_ln_g, conv_ln_b, norm_g, norm_b,
           qW, qb, kW, kb, vW, vb, oW, ob, op1_W, op1_b, op_ln_g, op_ln_b,
           op2_W, op2_b):
    src = edge_index[0].astype(jnp.int32)
    dst = edge_index[1].astype(jnp.int32)
    rel = edge_type.astype(jnp.int32)

    pad = EPAD - E
    pad_ar = jnp.arange(pad, dtype=jnp.int32)
    src_p = jnp.concatenate([src, pad_ar % 128])
    rel_p = jnp.concatenate([rel, jnp.zeros((pad,), jnp.int32)])
    dst_p = jnp.concatenate([dst, N + pad_ar % (NPAD - N)]).reshape(
        EPAD // CHUNK, CHUNK)
    cidx = _make_cidx(rel_p, src_p)
    zeros = jnp.zeros((NPAD, D), jnp.float32)

    h0, self0 = _t0(node_types, node_emb, type_emb, rel_W[0], rel_b[0],
                    self_W[0], self_b[0])
    parts0 = _sc_scatter(h0, cidx, dst_p, zeros)
    x1, h1, self1 = _fmid(self0, parts0, None, conv_ln_g[0], conv_ln_b[0],
                          norm_g[0], norm_b[0], rel_W[1], rel_b[1],
                          self_W[1], self_b[1], with_resid=False)
    parts1 = _sc_scatter(h1, cidx, dst_p, zeros)
    qkv1 = _proj(x1, qW, qb, kW, kb, vW, vb)
    x2, h2, self2 = _fmid(self1, parts1, x1, conv_ln_g[1], conv_ln_b[1],
                          norm_g[1], norm_b[1], rel_W[2], rel_b[2],
                          self_W[2], self_b[2], with_resid=True)
    parts2 = _sc_scatter(h2, cidx, dst_p, zeros)
    qkv2 = _proj(x2, qW, qb, kW, kb, vW, vb)
    return _readout(self2, parts2, x2, qkv1, qkv2, conv_ln_g[2],
                    conv_ln_b[2], norm_g[2], norm_b[2], qW, qb, kW, kb, vW,
                    vb, oW, ob, op1_W, op1_b, op_ln_g, op_ln_b, op2_W, op2_b)
